# Initial kernel scaffold; baseline (speedup 1.0000x reference)
import jax
import jax.numpy as jnp
from jax.experimental import pallas as pl


def kernel(node_rep, edge_rep, edge_index, W_lvl1, W_lvl2, W_lift, eps1, eps2):
    raise NotImplementedError("write your pallas kernel here")


# speedup vs baseline: 2.9446x; 2.9446x over previous
import functools

import jax
import jax.numpy as jnp
from jax import lax
from jax.experimental import pallas as pl
from jax.experimental.pallas import tpu as pltpu
from jax.experimental.pallas import tpu_sc as plsc

N = 10000
E = 320000
H = 128

NW = 32
NB_REAL = 2 * E // 128
BPW = 160
NB_PAD = NW * BPW
EP2 = NB_PAD * 128
GRP = 4

_mesh = plsc.VectorSubcoreMesh(core_axis_name="c", subcore_axis_name="s")


@functools.partial(
    pl.kernel,
    mesh=_mesh,
    out_type=jax.ShapeDtypeStruct((EP2, H), jnp.float32),
    scratch_types=[
        pltpu.VMEM((BPW, 128), jnp.int32),
        pltpu.VMEM((GRP * 128, H), jnp.float32),
        pltpu.SemaphoreType.DMA,
    ],
)
def _gather_sc(node_hbm, idx_hbm, out_hbm, idx_v, rows_v, sem):
    c = lax.axis_index("c")
    s = lax.axis_index("s")
    wid = s * 2 + c
    pltpu.sync_copy(idx_hbm.at[wid], idx_v)

    def body(j2, carry):
        gb0 = wid * BPW + j2 * GRP

        @pl.when(gb0 < NB_REAL)
        def _():
            handles = []
            for b in range(GRP):
                handles.append(
                    pltpu.async_copy(
                        node_hbm.at[idx_v.at[j2 * GRP + b]],
                        rows_v.at[pl.ds(b * 128, 128)],
                        sem,
                    )
                )
            for h in handles:
                h.wait()
            pltpu.sync_copy(rows_v, out_hbm.at[pl.ds(gb0 * 128, GRP * 128)])

        return carry

    lax.fori_loop(0, BPW // GRP, body, 0)


ACC_ROWS = N + 8
HH = H // 2

@functools.partial(
    pl.kernel,
    mesh=_mesh,
    out_type=jax.ShapeDtypeStruct((2, N, H), jnp.float32),
    scratch_types=[
        pltpu.VMEM((BPW, 128), jnp.int32),
        pltpu.VMEM((GRP * 128, HH), jnp.float32),
        pltpu.VMEM_SHARED((ACC_ROWS, HH), jnp.float32),
    ],
    compiler_params=pltpu.CompilerParams(use_tc_tiling_on_sc=False),
)
def _scatter_sc(eh_hbm, idx_hbm, zeros_hbm, out_hbm, idx_v, vals_v, acc_sh):
    c = lax.axis_index("c")
    s = lax.axis_index("s")
    wid = s * 2 + c

    pltpu.sync_copy(idx_hbm.at[wid], idx_v)

    for p in range(2):
        @pl.when(s < 15)
        def _():
            pltpu.sync_copy(zeros_hbm.at[pl.ds(0, 624)],
                            acc_sh.at[pl.ds(s * 624, 624)])

        @pl.when(s == 15)
        def _():
            pltpu.sync_copy(zeros_hbm, acc_sh.at[pl.ds(15 * 624, 640)])

        plsc.subcore_barrier()

        def body(j2, carry):
            gb0 = wid * BPW + j2 * GRP

            @pl.when(gb0 < NB_REAL)
            def _():
                vbase = jnp.where(gb0 < NB_REAL // 2,
                                  gb0 * 128,
                                  (gb0 - NB_REAL // 2) * 128)
                pltpu.sync_copy(
                    eh_hbm.at[pl.ds(vbase, GRP * 128), pl.ds(p * HH, HH)],
                    vals_v)
                for b in range(GRP):
                    pltpu.sync_copy(
                        vals_v.at[pl.ds(b * 128, 128)],
                        acc_sh.at[idx_v.at[j2 * GRP + b]],
                        add=True,
                    )

            return carry

        lax.fori_loop(0, BPW // GRP, body, 0)
        plsc.subcore_barrier()

        @pl.when(s < 15)
        def _():
            pltpu.sync_copy(acc_sh.at[pl.ds(s * 624, 624)],
                            out_hbm.at[c, pl.ds(s * 624, 624), pl.ds(p * HH, HH)])

        @pl.when(s == 15)
        def _():
            pltpu.sync_copy(acc_sh.at[pl.ds(15 * 624, 640)],
                            out_hbm.at[c, pl.ds(15 * 624, 640), pl.ds(p * HH, HH)])

        if p == 0:
            plsc.subcore_barrier()


BE = 512


def _edge_tc_body(eps2_ref, gs_ref, gd_ref, er_ref, w1a_ref, w1b_ref, wl_ref,
                  eh_ref, eo_ref):
    a = gs_ref[...] + gd_ref[...]
    eh = jnp.maximum(
        jnp.dot(a, w1a_ref[...], preferred_element_type=jnp.float32)
        + jnp.dot(er_ref[...], w1b_ref[...], preferred_element_type=jnp.float32),
        0.0)
    eh_ref[...] = eh
    t = (1.0 + eps2_ref[0]) * eh + a
    eo_ref[...] = jnp.maximum(
        jnp.dot(t, wl_ref[...], preferred_element_type=jnp.float32), 0.0)


def _edge_tc(g2, edge_rep, w1a, w1b, wl, eps2):
    grid = (E // BE,)
    return pl.pallas_call(
        _edge_tc_body,
        grid=grid,
        in_specs=[
            pl.BlockSpec(memory_space=pltpu.SMEM),
            pl.BlockSpec((BE, H), lambda i: (i, 0)),
            pl.BlockSpec((BE, H), lambda i: (i + E // BE, 0)),
            pl.BlockSpec((BE, H), lambda i: (i, 0)),
            pl.BlockSpec((H, H), lambda i: (0, 0)),
            pl.BlockSpec((H, H), lambda i: (0, 0)),
            pl.BlockSpec((H, H), lambda i: (0, 0)),
        ],
        out_specs=[
            pl.BlockSpec((BE, H), lambda i: (i, 0)),
            pl.BlockSpec((BE, H), lambda i: (i, 0)),
        ],
        out_shape=[
            jax.ShapeDtypeStruct((E, H), jnp.float32),
            jax.ShapeDtypeStruct((E, H), jnp.float32),
        ],
        compiler_params=pltpu.CompilerParams(
            dimension_semantics=("arbitrary",)),
    )(eps2, g2, g2, edge_rep, w1a, w1b, wl)


BN = 1000


def _node_tc_body(eps1_ref, nr_ref, a0_ref, a1_ref, w2_ref, out_ref):
    x = (1.0 + eps1_ref[0]) * nr_ref[...] + a0_ref[...] + a1_ref[...]
    out_ref[...] = jnp.maximum(
        jnp.dot(x, w2_ref[...], preferred_element_type=jnp.float32), 0.0)


def _node_tc(node_rep, a0, a1, w2, eps1):
    return pl.pallas_call(
        _node_tc_body,
        grid=(N // BN,),
        in_specs=[
            pl.BlockSpec(memory_space=pltpu.SMEM),
            pl.BlockSpec((BN, H), lambda i: (i, 0)),
            pl.BlockSpec((BN, H), lambda i: (i, 0)),
            pl.BlockSpec((BN, H), lambda i: (i, 0)),
            pl.BlockSpec((H, H), lambda i: (0, 0)),
        ],
        out_specs=pl.BlockSpec((BN, H), lambda i: (i, 0)),
        out_shape=jax.ShapeDtypeStruct((N, H), jnp.float32),
        compiler_params=pltpu.CompilerParams(
            dimension_semantics=("arbitrary",)),
    )(eps1, node_rep, a0, a1, w2)


def kernel(node_rep, edge_rep, edge_index, W_lvl1, W_lvl2, W_lift, eps1, eps2):
    src = edge_index[0]
    dst = edge_index[1]
    idx_cat = jnp.concatenate([src, dst])
    pad = NB_PAD * 128 - 2 * E
    idx_g = jnp.pad(idx_cat, (0, pad)).reshape(NW, BPW, 128)
    idx_s = jnp.pad(idx_cat, (0, pad), constant_values=N).reshape(NW, BPW, 128)

    g2 = _gather_sc(node_rep, idx_g)

    w1a = W_lvl1[:H]
    w1b = W_lvl1[H:]
    eh, eo = _edge_tc(g2, edge_rep, w1a, w1b, W_lift,
                      jnp.reshape(eps2, (1,)))

    zeros = jnp.zeros((640, HH), jnp.float32)
    parts = _scatter_sc(eh, idx_s, zeros)

    node_out = _node_tc(node_rep, parts[0], parts[1], W_lvl2,
                        jnp.reshape(eps1, (1,)))
    return node_out, eo


# speedup vs baseline: 3.3242x; 1.1289x over previous
import functools

import jax
import jax.numpy as jnp
from jax import lax
from jax.experimental import pallas as pl
from jax.experimental.pallas import tpu as pltpu
from jax.experimental.pallas import tpu_sc as plsc

N = 10000
E = 320000
H = 128

NW = 32
NB_REAL = 2 * E // 128
GBPW = 160
EP2 = NW * GBPW * 128
GGRP = 2

SB_REAL = E // 128
SBPW = 80
EPAD = NW * SBPW * 128
SGRP = 4

_mesh = plsc.VectorSubcoreMesh(core_axis_name="c", subcore_axis_name="s")


@functools.partial(
    pl.kernel,
    mesh=_mesh,
    out_type=jax.ShapeDtypeStruct((EP2, H), jnp.float32),
    scratch_types=[
        pltpu.VMEM((GBPW, 128), jnp.int32),
        pltpu.VMEM((2, GGRP * 128, H), jnp.float32),
        pltpu.SemaphoreType.DMA,
        pltpu.SemaphoreType.DMA,
    ],
)
def _gather_sc(node_hbm, idx_hbm, out_hbm, idx_v, rows_v, gsem, ssem):
    c = lax.axis_index("c")
    s = lax.axis_index("s")
    wid = s * 2 + c
    pltpu.sync_copy(idx_hbm.at[wid], idx_v)

    n_grp = GBPW // GGRP

    def body(j2, carry):
        gb0 = wid * GBPW + j2 * GGRP

        @pl.when(gb0 < NB_REAL)
        def _():
            slot = j2 % 2

            @pl.when(j2 >= 2)
            def _():
                pltpu.make_async_copy(
                    rows_v.at[0], out_hbm.at[pl.ds(0, GGRP * 128)], ssem
                ).wait()

            handles = []
            for b in range(GGRP):
                handles.append(
                    pltpu.async_copy(
                        node_hbm.at[idx_v.at[j2 * GGRP + b]],
                        rows_v.at[slot, pl.ds(b * 128, 128)],
                        gsem,
                    )
                )
            for h in handles:
                h.wait()
            pltpu.async_copy(
                rows_v.at[slot],
                out_hbm.at[pl.ds(gb0 * 128, GGRP * 128)],
                ssem,
            )

        return carry

    lax.fori_loop(0, n_grp, body, 0)
    for _ in range(2):
        pltpu.make_async_copy(
            rows_v.at[0], out_hbm.at[pl.ds(0, GGRP * 128)], ssem
        ).wait()


ACC_ROWS = N + 8
HH = H // 2

@functools.partial(
    pl.kernel,
    mesh=_mesh,
    out_type=jax.ShapeDtypeStruct((2, N, H), jnp.float32),
    scratch_types=[
        pltpu.VMEM((SBPW, 128), jnp.int32),
        pltpu.VMEM((SBPW, 128), jnp.int32),
        pltpu.VMEM((2, SGRP * 128, HH), jnp.float32),
        pltpu.VMEM_SHARED((ACC_ROWS, HH), jnp.float32),
        pltpu.SemaphoreType.DMA,
    ],
    compiler_params=pltpu.CompilerParams(use_tc_tiling_on_sc=False),
)
def _scatter_sc(eh_hbm, src_hbm, dst_hbm, zeros_hbm, out_hbm,
                sidx_v, didx_v, vals_v, acc_sh, scsem):
    c = lax.axis_index("c")
    s = lax.axis_index("s")
    wid = s * 2 + c

    pltpu.sync_copy(src_hbm.at[wid], sidx_v)
    pltpu.sync_copy(dst_hbm.at[wid], didx_v)

    n_grp = SBPW // SGRP
    ops_per_grp = 2 * SGRP

    def drain_one():
        pltpu.make_async_copy(
            vals_v.at[0, pl.ds(0, 128)], acc_sh.at[sidx_v.at[0]], scsem
        ).wait()

    for p in range(2):
        @pl.when(s < 15)
        def _():
            pltpu.sync_copy(zeros_hbm.at[pl.ds(0, 624)],
                            acc_sh.at[pl.ds(s * 624, 624)])

        @pl.when(s == 15)
        def _():
            pltpu.sync_copy(zeros_hbm, acc_sh.at[pl.ds(15 * 624, 640)])

        plsc.subcore_barrier()

        def body(j2, carry):
            gb0 = wid * SBPW + j2 * SGRP

            @pl.when(gb0 < SB_REAL)
            def _():
                slot = j2 % 2

                @pl.when(j2 >= 2)
                def _():
                    for _ in range(ops_per_grp):
                        drain_one()

                pltpu.sync_copy(
                    eh_hbm.at[pl.ds(gb0 * 128, SGRP * 128), pl.ds(p * HH, HH)],
                    vals_v.at[slot])
                for b in range(SGRP):
                    v = vals_v.at[slot, pl.ds(b * 128, 128)]
                    pltpu.async_copy(v, acc_sh.at[sidx_v.at[j2 * SGRP + b]],
                                     scsem, add=True)
                    pltpu.async_copy(v, acc_sh.at[didx_v.at[j2 * SGRP + b]],
                                     scsem, add=True)

            return carry

        lax.fori_loop(0, n_grp, body, 0)
        for _ in range(2 * ops_per_grp):
            drain_one()
        plsc.subcore_barrier()

        @pl.when(s < 15)
        def _():
            pltpu.sync_copy(acc_sh.at[pl.ds(s * 624, 624)],
                            out_hbm.at[c, pl.ds(s * 624, 624), pl.ds(p * HH, HH)])

        @pl.when(s == 15)
        def _():
            pltpu.sync_copy(acc_sh.at[pl.ds(15 * 624, 640)],
                            out_hbm.at[c, pl.ds(15 * 624, 640), pl.ds(p * HH, HH)])

        if p == 0:
            plsc.subcore_barrier()


BE = 512


def _edge_tc_body(eps2_ref, gs_ref, gd_ref, er_ref, w1a_ref, w1b_ref, wl_ref,
                  eh_ref, eo_ref):
    a = gs_ref[...] + gd_ref[...]
    eh = jnp.maximum(
        jnp.dot(a.astype(jnp.bfloat16), w1a_ref[...].astype(jnp.bfloat16),
                preferred_element_type=jnp.float32)
        + jnp.dot(er_ref[...].astype(jnp.bfloat16),
                  w1b_ref[...].astype(jnp.bfloat16),
                  preferred_element_type=jnp.float32),
        0.0)
    eh_ref[...] = eh
    t = (1.0 + eps2_ref[0]) * eh + a
    eo_ref[...] = jnp.maximum(
        jnp.dot(t.astype(jnp.bfloat16), wl_ref[...].astype(jnp.bfloat16),
                preferred_element_type=jnp.float32),
        0.0)


def _edge_tc(g2, edge_rep, w1a, w1b, wl, eps2):
    grid = (E // BE,)
    return pl.pallas_call(
        _edge_tc_body,
        grid=grid,
        in_specs=[
            pl.BlockSpec(memory_space=pltpu.SMEM),
            pl.BlockSpec((BE, H), lambda i: (i, 0)),
            pl.BlockSpec((BE, H), lambda i: (i + E // BE, 0)),
            pl.BlockSpec((BE, H), lambda i: (i, 0)),
            pl.BlockSpec((H, H), lambda i: (0, 0)),
            pl.BlockSpec((H, H), lambda i: (0, 0)),
            pl.BlockSpec((H, H), lambda i: (0, 0)),
        ],
        out_specs=[
            pl.BlockSpec((BE, H), lambda i: (i, 0)),
            pl.BlockSpec((BE, H), lambda i: (i, 0)),
        ],
        out_shape=[
            jax.ShapeDtypeStruct((EPAD, H), jnp.float32),
            jax.ShapeDtypeStruct((E, H), jnp.float32),
        ],
        compiler_params=pltpu.CompilerParams(
            dimension_semantics=("arbitrary",)),
    )(eps2, g2, g2, edge_rep, w1a, w1b, wl)


BN = 1000


def _node_tc_body(eps1_ref, nr_ref, a0_ref, a1_ref, w2_ref, out_ref):
    x = (1.0 + eps1_ref[0]) * nr_ref[...] + a0_ref[...] + a1_ref[...]
    out_ref[...] = jnp.maximum(
        jnp.dot(x, w2_ref[...], preferred_element_type=jnp.float32), 0.0)


def _node_tc(node_rep, a0, a1, w2, eps1):
    return pl.pallas_call(
        _node_tc_body,
        grid=(N // BN,),
        in_specs=[
            pl.BlockSpec(memory_space=pltpu.SMEM),
            pl.BlockSpec((BN, H), lambda i: (i, 0)),
            pl.BlockSpec((BN, H), lambda i: (i, 0)),
            pl.BlockSpec((BN, H), lambda i: (i, 0)),
            pl.BlockSpec((H, H), lambda i: (0, 0)),
        ],
        out_specs=pl.BlockSpec((BN, H), lambda i: (i, 0)),
        out_shape=jax.ShapeDtypeStruct((N, H), jnp.float32),
        compiler_params=pltpu.CompilerParams(
            dimension_semantics=("arbitrary",)),
    )(eps1, node_rep, a0, a1, w2)


def kernel(node_rep, edge_rep, edge_index, W_lvl1, W_lvl2, W_lift, eps1, eps2):
    src = edge_index[0]
    dst = edge_index[1]
    idx_cat = jnp.concatenate([src, dst])
    idx_g = jnp.pad(idx_cat, (0, EP2 - 2 * E)).reshape(NW, GBPW, 128)
    src_s = jnp.pad(src, (0, EPAD - E), constant_values=N).reshape(NW, SBPW, 128)
    dst_s = jnp.pad(dst, (0, EPAD - E), constant_values=N).reshape(NW, SBPW, 128)

    g2 = _gather_sc(node_rep, idx_g)

    w1a = W_lvl1[:H]
    w1b = W_lvl1[H:]
    eh, eo = _edge_tc(g2, edge_rep, w1a, w1b, W_lift,
                      jnp.reshape(eps2, (1,)))

    zeros = jnp.zeros((640, HH), jnp.float32)
    parts = _scatter_sc(eh, src_s, dst_s, zeros)

    node_out = _node_tc(node_rep, parts[0], parts[1], W_lvl2,
                        jnp.reshape(eps1, (1,)))
    return node_out, eo


# speedup vs baseline: 4.5230x; 1.3606x over previous
import functools

import jax
import jax.numpy as jnp
from jax import lax
from jax.experimental import pallas as pl
from jax.experimental.pallas import tpu as pltpu
from jax.experimental.pallas import tpu_sc as plsc

N = 10000
E = 320000
H = 128

NW = 32
NB_REAL = 2 * E // 128
GBPW = 160
EP2 = NW * GBPW * 128
GGRP = 2

SB_REAL = E // 128
SBPW = 80
EPAD = NW * SBPW * 128
SGRP = 4

_mesh = plsc.VectorSubcoreMesh(core_axis_name="c", subcore_axis_name="s")


@functools.partial(
    pl.kernel,
    mesh=_mesh,
    out_type=jax.ShapeDtypeStruct((EP2, H), jnp.float32),
    scratch_types=[
        pltpu.VMEM((GBPW, 128), jnp.int32),
        pltpu.VMEM((2, GGRP * 128, H), jnp.float32),
        pltpu.SemaphoreType.DMA,
        pltpu.SemaphoreType.DMA,
    ],
)
def _gather_sc(node_hbm, idx_hbm, out_hbm, idx_v, rows_v, gsem, ssem):
    c = lax.axis_index("c")
    s = lax.axis_index("s")
    wid = s * 2 + c
    pltpu.sync_copy(idx_hbm.at[wid], idx_v)

    n_grp = GBPW // GGRP

    def body(j2, carry):
        gb0 = wid * GBPW + j2 * GGRP

        @pl.when(gb0 < NB_REAL)
        def _():
            slot = j2 % 2

            @pl.when(j2 >= 2)
            def _():
                pltpu.make_async_copy(
                    rows_v.at[0], out_hbm.at[pl.ds(0, GGRP * 128)], ssem
                ).wait()

            handles = []
            for b in range(GGRP):
                handles.append(
                    pltpu.async_copy(
                        node_hbm.at[idx_v.at[j2 * GGRP + b]],
                        rows_v.at[slot, pl.ds(b * 128, 128)],
                        gsem,
                    )
                )
            for h in handles:
                h.wait()
            pltpu.async_copy(
                rows_v.at[slot],
                out_hbm.at[pl.ds(gb0 * 128, GGRP * 128)],
                ssem,
            )

        return carry

    lax.fori_loop(0, n_grp, body, 0)
    for _ in range(2):
        pltpu.make_async_copy(
            rows_v.at[0], out_hbm.at[pl.ds(0, GGRP * 128)], ssem
        ).wait()


ACC_ROWS = N + 8
HH = H // 2

@functools.partial(
    pl.kernel,
    mesh=_mesh,
    out_type=jax.ShapeDtypeStruct((2, N, H), jnp.float32),
    scratch_types=[
        pltpu.VMEM((SBPW, 128), jnp.int32),
        pltpu.VMEM((SBPW, 128), jnp.int32),
        pltpu.VMEM((2, SGRP * 128, HH), jnp.float32),
        pltpu.VMEM_SHARED((ACC_ROWS, HH), jnp.float32),
        pltpu.SemaphoreType.DMA,
    ],
    compiler_params=pltpu.CompilerParams(use_tc_tiling_on_sc=False),
)
def _scatter_sc(eh_hbm, src_hbm, dst_hbm, zeros_hbm, out_hbm,
                sidx_v, didx_v, vals_v, acc_sh, scsem):
    c = lax.axis_index("c")
    s = lax.axis_index("s")
    wid = s * 2 + c

    pltpu.sync_copy(src_hbm.at[wid], sidx_v)
    pltpu.sync_copy(dst_hbm.at[wid], didx_v)

    n_grp = SBPW // SGRP
    ops_per_grp = 2 * SGRP

    def drain_one():
        pltpu.make_async_copy(
            vals_v.at[0, pl.ds(0, 128)], acc_sh.at[sidx_v.at[0]], scsem
        ).wait()

    for p in range(2):
        @pl.when(s < 15)
        def _():
            pltpu.sync_copy(zeros_hbm.at[pl.ds(0, 624)],
                            acc_sh.at[pl.ds(s * 624, 624)])

        @pl.when(s == 15)
        def _():
            pltpu.sync_copy(zeros_hbm, acc_sh.at[pl.ds(15 * 624, 640)])

        plsc.subcore_barrier()

        def body(j2, carry):
            gb0 = wid * SBPW + j2 * SGRP

            @pl.when(gb0 < SB_REAL)
            def _():
                slot = j2 % 2

                @pl.when(j2 >= 2)
                def _():
                    for _ in range(ops_per_grp):
                        drain_one()

                pltpu.sync_copy(
                    eh_hbm.at[pl.ds(gb0 * 128, SGRP * 128), pl.ds(p * HH, HH)],
                    vals_v.at[slot])
                for b in range(SGRP):
                    v = vals_v.at[slot, pl.ds(b * 128, 128)]
                    pltpu.async_copy(v, acc_sh.at[sidx_v.at[j2 * SGRP + b]],
                                     scsem, add=True)
                    pltpu.async_copy(v, acc_sh.at[didx_v.at[j2 * SGRP + b]],
                                     scsem, add=True)

            return carry

        lax.fori_loop(0, n_grp, body, 0)
        for _ in range(2 * ops_per_grp):
            drain_one()
        plsc.subcore_barrier()

        @pl.when(s < 15)
        def _():
            pltpu.sync_copy(acc_sh.at[pl.ds(s * 624, 624)],
                            out_hbm.at[c, pl.ds(s * 624, 624), pl.ds(p * HH, HH)])

        @pl.when(s == 15)
        def _():
            pltpu.sync_copy(acc_sh.at[pl.ds(15 * 624, 640)],
                            out_hbm.at[c, pl.ds(15 * 624, 640), pl.ds(p * HH, HH)])

        if p == 0:
            plsc.subcore_barrier()


BE = 2000


def _edge_tc_body(eps2_ref, gs_ref, gd_ref, er_ref, w1a_ref, w1b_ref, wl_ref,
                  eh_ref, eo_ref):
    a = gs_ref[...] + gd_ref[...]
    eh = jnp.maximum(
        jnp.dot(a.astype(jnp.bfloat16), w1a_ref[...].astype(jnp.bfloat16),
                preferred_element_type=jnp.float32)
        + jnp.dot(er_ref[...].astype(jnp.bfloat16),
                  w1b_ref[...].astype(jnp.bfloat16),
                  preferred_element_type=jnp.float32),
        0.0)
    eh_ref[...] = eh
    t = (1.0 + eps2_ref[0]) * eh + a
    eo_ref[...] = jnp.maximum(
        jnp.dot(t.astype(jnp.bfloat16), wl_ref[...].astype(jnp.bfloat16),
                preferred_element_type=jnp.float32),
        0.0)


def _edge_tc(g2, edge_rep, w1a, w1b, wl, eps2):
    grid = (E // BE,)
    return pl.pallas_call(
        _edge_tc_body,
        grid=grid,
        in_specs=[
            pl.BlockSpec(memory_space=pltpu.SMEM),
            pl.BlockSpec((BE, H), lambda i: (i, 0)),
            pl.BlockSpec((BE, H), lambda i: (i + E // BE, 0)),
            pl.BlockSpec((BE, H), lambda i: (i, 0)),
            pl.BlockSpec((H, H), lambda i: (0, 0)),
            pl.BlockSpec((H, H), lambda i: (0, 0)),
            pl.BlockSpec((H, H), lambda i: (0, 0)),
        ],
        out_specs=[
            pl.BlockSpec((BE, H), lambda i: (i, 0)),
            pl.BlockSpec((BE, H), lambda i: (i, 0)),
        ],
        out_shape=[
            jax.ShapeDtypeStruct((EPAD, H), jnp.float32),
            jax.ShapeDtypeStruct((E, H), jnp.float32),
        ],
        compiler_params=pltpu.CompilerParams(
            dimension_semantics=("arbitrary",)),
    )(eps2, g2, g2, edge_rep, w1a, w1b, wl)


BN = 1000


def _node_tc_body(eps1_ref, nr_ref, a0_ref, a1_ref, w2_ref, out_ref):
    x = (1.0 + eps1_ref[0]) * nr_ref[...] + a0_ref[...] + a1_ref[...]
    out_ref[...] = jnp.maximum(
        jnp.dot(x, w2_ref[...], preferred_element_type=jnp.float32), 0.0)


def _node_tc(node_rep, a0, a1, w2, eps1):
    return pl.pallas_call(
        _node_tc_body,
        grid=(N // BN,),
        in_specs=[
            pl.BlockSpec(memory_space=pltpu.SMEM),
            pl.BlockSpec((BN, H), lambda i: (i, 0)),
            pl.BlockSpec((BN, H), lambda i: (i, 0)),
            pl.BlockSpec((BN, H), lambda i: (i, 0)),
            pl.BlockSpec((H, H), lambda i: (0, 0)),
        ],
        out_specs=pl.BlockSpec((BN, H), lambda i: (i, 0)),
        out_shape=jax.ShapeDtypeStruct((N, H), jnp.float32),
        compiler_params=pltpu.CompilerParams(
            dimension_semantics=("arbitrary",)),
    )(eps1, node_rep, a0, a1, w2)


def kernel(node_rep, edge_rep, edge_index, W_lvl1, W_lvl2, W_lift, eps1, eps2):
    src = edge_index[0]
    dst = edge_index[1]
    idx_cat = jnp.concatenate([src, dst])
    idx_g = jnp.pad(idx_cat, (0, EP2 - 2 * E)).reshape(NW, GBPW, 128)
    src_s = jnp.pad(src, (0, EPAD - E), constant_values=N).reshape(NW, SBPW, 128)
    dst_s = jnp.pad(dst, (0, EPAD - E), constant_values=N).reshape(NW, SBPW, 128)

    g2 = _gather_sc(node_rep, idx_g)

    w1a = W_lvl1[:H]
    w1b = W_lvl1[H:]
    eh, eo = _edge_tc(g2, edge_rep, w1a, w1b, W_lift,
                      jnp.reshape(eps2, (1,)))

    zeros = jnp.zeros((640, HH), jnp.float32)
    parts = _scatter_sc(eh, src_s, dst_s, zeros)

    node_out = _node_tc(node_rep, parts[0], parts[1], W_lvl2,
                        jnp.reshape(eps1, (1,)))
    return node_out, eo
